# baseline (speedup 1.0000x reference)
import functools
import math

import jax
import jax.numpy as jnp
from jax import lax
from jax.experimental import pallas as pl
from jax.experimental.pallas import tpu as pltpu
from jax.experimental.pallas import tpu_sc as plsc

D_MODEL = 64
VOCAB = 1000000
BATCH = 4096
SEQ = 200
SCALE = math.sqrt(D_MODEL)

NC = 2
NS = 16
NW = NC * NS

B = BATCH * SEQ
B_PER_W = B // NW
IDX_MINOR = 128
CHUNK = 512
G_PER_CHUNK = CHUNK // IDX_MINOR
N_CHUNKS = B_PER_W // CHUNK
IDX_ROWS_PER_W = B_PER_W // IDX_MINOR


def _sc_body(idx_hbm, table_hbm, out_hbm, idx_v, rows_v, sem):
    wid = lax.axis_index("s") * NC + lax.axis_index("c")
    row_base = wid * IDX_ROWS_PER_W
    out_base = wid * B_PER_W

    def chunk_body(g, _):
        pltpu.sync_copy(
            idx_hbm.at[pl.ds(row_base + g * G_PER_CHUNK, G_PER_CHUNK)], idx_v
        )
        copies = []
        for j in range(G_PER_CHUNK):
            copies.append(
                pltpu.async_copy(
                    table_hbm.at[idx_v.at[j]],
                    rows_v.at[pl.ds(j * IDX_MINOR, IDX_MINOR)],
                    sem,
                )
            )
        for c in copies:
            c.wait()

        def scale_row(r, _):
            for c in range(D_MODEL // 16):
                sl = pl.ds(c * 16, 16)
                rows_v[r, sl] = rows_v[r, sl] * SCALE
            return 0

        lax.fori_loop(0, CHUNK, scale_row, 0)

        pltpu.sync_copy(rows_v, out_hbm.at[pl.ds(out_base + g * CHUNK, CHUNK)])
        return 0

    lax.fori_loop(0, N_CHUNKS, chunk_body, 0)


@jax.jit
def _embed(x_flat2d, lut_weight):
    mesh = plsc.VectorSubcoreMesh(core_axis_name="c", subcore_axis_name="s")
    out = pl.kernel(
        _sc_body,
        out_type=jax.ShapeDtypeStruct((B, D_MODEL), jnp.float32),
        mesh=mesh,
        scratch_types=[
            pltpu.VMEM((G_PER_CHUNK, IDX_MINOR), jnp.int32),
            pltpu.VMEM((CHUNK, D_MODEL), jnp.float32),
            pltpu.SemaphoreType.DMA,
        ],
        compiler_params=pltpu.CompilerParams(use_tc_tiling_on_sc=False),
    )(x_flat2d, lut_weight)
    return out.reshape(BATCH, SEQ, D_MODEL)


def kernel(x, lut_weight):
    x_flat2d = x.reshape(B // IDX_MINOR, IDX_MINOR)
    return _embed(x_flat2d, lut_weight)


# speedup vs baseline: 1.1343x; 1.1343x over previous
import math

import jax
import jax.numpy as jnp
from jax import lax
from jax.experimental import pallas as pl
from jax.experimental.pallas import tpu as pltpu
from jax.experimental.pallas import tpu_sc as plsc

D_MODEL = 64
VOCAB = 1000000
BATCH = 4096
SEQ = 200
SCALE = math.sqrt(D_MODEL)

NC = 2
NS = 16
NW = NC * NS

B = BATCH * SEQ
B_PER_W = B // NW
IDX_MINOR = 128
CHUNK = 512
G_PER_CHUNK = CHUNK // IDX_MINOR
N_CHUNKS = B_PER_W // CHUNK
IDX_ROWS_PER_W = B_PER_W // IDX_MINOR


def _sc_body(idx_hbm, table_hbm, out_hbm, idx_v, rows_v, s_g0, s_g1, s_o0, s_o1):
    s_g = (s_g0, s_g1)
    s_o = (s_o0, s_o1)
    wid = lax.axis_index("s") * NC + lax.axis_index("c")
    row_base = wid * IDX_ROWS_PER_W
    out_base = wid * B_PER_W

    def fire_gathers(g, b):
        for j in range(G_PER_CHUNK):
            pltpu.async_copy(
                table_hbm.at[idx_v.at[g * G_PER_CHUNK + j]],
                rows_v.at[b, pl.ds(j * IDX_MINOR, IDX_MINOR)],
                s_g[b],
            )

    def wait_gathers(b):
        for j in range(G_PER_CHUNK):
            pltpu.make_async_copy(
                table_hbm.at[idx_v.at[j]],
                rows_v.at[b, pl.ds(j * IDX_MINOR, IDX_MINOR)],
                s_g[b],
            ).wait()

    def wait_store(b):
        pltpu.make_async_copy(
            rows_v.at[b], out_hbm.at[pl.ds(out_base, CHUNK)], s_o[b]
        ).wait()

    pltpu.sync_copy(idx_hbm.at[pl.ds(row_base, IDX_ROWS_PER_W)], idx_v)
    fire_gathers(0, 0)

    def outer(gg, _):
        for b in range(2):
            g = gg + b
            nb = 1 - b

            @pl.when(g + 1 < N_CHUNKS)
            def _fire_next():
                @pl.when(g >= 1)
                def _drain_prev_store():
                    wait_store(nb)

                fire_gathers(g + 1, nb)

            wait_gathers(b)

            @plsc.parallel_loop(0, CHUNK, unroll=8)
            def _scale(r):
                for c in range(D_MODEL // 16):
                    sl = pl.ds(c * 16, 16)
                    rows_v[b, r, sl] = rows_v[b, r, sl] * SCALE

            pltpu.async_copy(
                rows_v.at[b], out_hbm.at[pl.ds(out_base + g * CHUNK, CHUNK)], s_o[b]
            )
        return 0

    lax.fori_loop(0, N_CHUNKS // 2, lambda i, c: outer(i * 2, c), 0)
    wait_store(0)
    wait_store(1)


@jax.jit
def _embed(x_flat2d, lut_weight):
    mesh = plsc.VectorSubcoreMesh(core_axis_name="c", subcore_axis_name="s")
    out = pl.kernel(
        _sc_body,
        out_type=jax.ShapeDtypeStruct((B, D_MODEL), jnp.float32),
        mesh=mesh,
        scratch_types=[
            pltpu.VMEM((IDX_ROWS_PER_W, IDX_MINOR), jnp.int32),
            pltpu.VMEM((2, CHUNK, D_MODEL), jnp.float32),
            pltpu.SemaphoreType.DMA,
            pltpu.SemaphoreType.DMA,
            pltpu.SemaphoreType.DMA,
            pltpu.SemaphoreType.DMA,
        ],
        compiler_params=pltpu.CompilerParams(use_tc_tiling_on_sc=False),
    )(x_flat2d, lut_weight)
    return out.reshape(BATCH, SEQ, D_MODEL)


def kernel(x, lut_weight):
    x_flat2d = x.reshape(B // IDX_MINOR, IDX_MINOR)
    return _embed(x_flat2d, lut_weight)
